# Initial kernel scaffold; baseline (speedup 1.0000x reference)
import jax
import jax.numpy as jnp
from jax.experimental import pallas as pl


def kernel(features, labels, prototypes):
    raise NotImplementedError("write your pallas kernel here")


# speedup vs baseline: 1.1382x; 1.1382x over previous
import functools

import jax
import jax.numpy as jnp
from jax import lax
from jax.experimental import pallas as pl
from jax.experimental.pallas import tpu as pltpu

_BLK = 4096
_CPAD = 128


def _loss_body(nblk, c, f_ref, l_ref, pt_ref, out_ref, s_acc, qc_acc):
    bi = pl.program_id(0)
    ji = pl.program_id(1)
    step = bi * nblk + ji
    nsteps = pl.num_programs(0) * pl.num_programs(1)

    @pl.when(step == 0)
    def _init():
        s_acc[...] = jnp.zeros_like(s_acc)
        qc_acc[...] = jnp.zeros_like(qc_acc)

    f = f_ref[0]
    labs = l_ref[0]

    klass = lax.broadcasted_iota(jnp.int32, (_CPAD, _BLK), 0)
    m = (klass == labs).astype(jnp.float32)

    s_acc[...] += lax.dot_general(
        f, m, (((1,), (1,)), ((), ())),
        preferred_element_type=jnp.float32,
        precision=lax.Precision.HIGHEST)

    rowsq = jnp.sum(f * f, axis=0, keepdims=True)
    r = jnp.concatenate(
        [rowsq, jnp.ones_like(rowsq), jnp.zeros((6, _BLK), jnp.float32)],
        axis=0)
    qc_acc[...] += lax.dot_general(
        r, m, (((1,), (1,)), ((), ())),
        preferred_element_type=jnp.float32,
        precision=lax.Precision.HIGHEST)

    @pl.when(step == nsteps - 1)
    def _finish():
        s = s_acc[...]
        q = qc_acc[0:1, :]
        n = qc_acc[1:2, :]
        pt = pt_ref[...]
        ps = jnp.sum(pt * s, axis=0, keepdims=True)
        pp = jnp.sum(pt * pt, axis=0, keepdims=True)
        present = n > 0.0
        denom = jnp.where(present, n, 1.0) * jnp.float32(c)
        term = jnp.where(present, (q + n * pp - 2.0 * ps) / denom, 0.0)
        loss = jnp.sum(term) / jnp.sum(present.astype(jnp.float32))
        out_ref[0, 0] = loss


def kernel(features, labels, prototypes):
    b, c, h, w = features.shape
    ncls = prototypes.shape[0]
    npix = h * w
    nblk = npix // _BLK

    feats = features.reshape(b, c, npix)
    labs = labels.astype(jnp.int32).reshape(b, 1, npix)
    pt = jnp.zeros((c, _CPAD), jnp.float32).at[:, :ncls].set(prototypes.T)

    out = pl.pallas_call(
        functools.partial(_loss_body, nblk, c),
        grid=(b, nblk),
        in_specs=[
            pl.BlockSpec((1, c, _BLK), lambda i, j: (i, 0, j)),
            pl.BlockSpec((1, 1, _BLK), lambda i, j: (i, 0, j)),
            pl.BlockSpec((c, _CPAD), lambda i, j: (0, 0)),
        ],
        out_specs=pl.BlockSpec(memory_space=pltpu.SMEM),
        out_shape=jax.ShapeDtypeStruct((1, 1), jnp.float32),
        scratch_shapes=[
            pltpu.VMEM((c, _CPAD), jnp.float32),
            pltpu.VMEM((8, _CPAD), jnp.float32),
        ],
    )(feats, labs, pt)
    return out.reshape(())


# speedup vs baseline: 2.2434x; 1.9710x over previous
import functools

import jax
import jax.numpy as jnp
from jax import lax
from jax.experimental import pallas as pl
from jax.experimental.pallas import tpu as pltpu

_BLK = 4096
_CPAD = 32


def _loss_body(nblk, c, f_ref, l_ref, pt_ref, out_ref, acc):
    bi = pl.program_id(0)
    ji = pl.program_id(1)
    step = bi * nblk + ji
    nsteps = pl.num_programs(0) * pl.num_programs(1)

    @pl.when(step == 0)
    def _init():
        acc[...] = jnp.zeros_like(acc)

    f = f_ref[0]
    labs = l_ref[0]

    klass = lax.broadcasted_iota(jnp.int32, (_CPAD, _BLK), 0)
    m = (klass == labs).astype(jnp.float32)

    g = jnp.concatenate(
        [f, f * f, jnp.ones((8, _BLK), jnp.float32)], axis=0)
    acc[...] += lax.dot_general(
        g, m, (((1,), (1,)), ((), ())),
        preferred_element_type=jnp.float32)

    @pl.when(step == nsteps - 1)
    def _finish():
        s = acc[0:c, :]
        q = jnp.sum(acc[c:2 * c, :], axis=0, keepdims=True)
        n = acc[2 * c:2 * c + 1, :]
        pt = pt_ref[...]
        ps = jnp.sum(pt * s, axis=0, keepdims=True)
        pp = jnp.sum(pt * pt, axis=0, keepdims=True)
        present = n > 0.0
        denom = jnp.where(present, n, 1.0) * jnp.float32(c)
        term = jnp.where(present, (q + n * pp - 2.0 * ps) / denom, 0.0)
        loss = jnp.sum(term) / jnp.sum(present.astype(jnp.float32))
        out_ref[0, 0] = loss


def kernel(features, labels, prototypes):
    b, c, h, w = features.shape
    ncls = prototypes.shape[0]
    npix = h * w
    nblk = npix // _BLK

    feats = features.reshape(b, c, npix)
    labs = labels.astype(jnp.int32).reshape(b, 1, npix)
    pt = jnp.zeros((c, _CPAD), jnp.float32).at[:, :ncls].set(prototypes.T)

    out = pl.pallas_call(
        functools.partial(_loss_body, nblk, c),
        grid=(b, nblk),
        in_specs=[
            pl.BlockSpec((1, c, _BLK), lambda i, j: (i, 0, j)),
            pl.BlockSpec((1, 1, _BLK), lambda i, j: (i, 0, j)),
            pl.BlockSpec((c, _CPAD), lambda i, j: (0, 0)),
        ],
        out_specs=pl.BlockSpec(memory_space=pltpu.SMEM),
        out_shape=jax.ShapeDtypeStruct((1, 1), jnp.float32),
        scratch_shapes=[
            pltpu.VMEM((2 * c + 8, _CPAD), jnp.float32),
        ],
    )(feats, labs, pt)
    return out.reshape(())


# speedup vs baseline: 2.5931x; 1.1559x over previous
import functools

import jax
import jax.numpy as jnp
from jax import lax
from jax.experimental import pallas as pl
from jax.experimental.pallas import tpu as pltpu

_BLK = 8192
_CPAD = 32


def _loss_body(nblk, c, f_ref, l_ref, pt_ref, out_ref, acc):
    bi = pl.program_id(0)
    ji = pl.program_id(1)
    step = bi * nblk + ji
    nsteps = pl.num_programs(0) * pl.num_programs(1)

    @pl.when(step == 0)
    def _init():
        acc[...] = jnp.zeros_like(acc)

    f = f_ref[0]
    labs = l_ref[0]

    klass = lax.broadcasted_iota(jnp.int32, (_CPAD, _BLK), 0)
    m = (klass == labs).astype(jnp.float32)

    rowsq = jnp.sum(f * f, axis=0, keepdims=True)
    g = jnp.concatenate(
        [f, rowsq, jnp.ones_like(rowsq)], axis=0)
    acc[...] += lax.dot_general(
        g, m, (((1,), (1,)), ((), ())),
        preferred_element_type=jnp.float32)

    @pl.when(step == nsteps - 1)
    def _finish():
        s = acc[0:c, :]
        q = acc[c:c + 1, :]
        n = acc[c + 1:c + 2, :]
        pt = pt_ref[...]
        ps = jnp.sum(pt * s, axis=0, keepdims=True)
        pp = jnp.sum(pt * pt, axis=0, keepdims=True)
        present = n > 0.0
        denom = jnp.where(present, n, 1.0) * jnp.float32(c)
        term = jnp.where(present, (q + n * pp - 2.0 * ps) / denom, 0.0)
        loss = jnp.sum(term) / jnp.sum(present.astype(jnp.float32))
        out_ref[0, 0] = loss


def kernel(features, labels, prototypes):
    b, c, h, w = features.shape
    ncls = prototypes.shape[0]
    npix = h * w
    nblk = npix // _BLK

    feats = features.reshape(b, c, npix)
    labs = labels.astype(jnp.int32).reshape(b, 1, npix)
    pt = jnp.zeros((c, _CPAD), jnp.float32).at[:, :ncls].set(prototypes.T)

    out = pl.pallas_call(
        functools.partial(_loss_body, nblk, c),
        grid=(b, nblk),
        in_specs=[
            pl.BlockSpec((1, c, _BLK), lambda i, j: (i, 0, j)),
            pl.BlockSpec((1, 1, _BLK), lambda i, j: (i, 0, j)),
            pl.BlockSpec((c, _CPAD), lambda i, j: (0, 0)),
        ],
        out_specs=pl.BlockSpec(memory_space=pltpu.SMEM),
        out_shape=jax.ShapeDtypeStruct((1, 1), jnp.float32),
        scratch_shapes=[
            pltpu.VMEM((c + 2, _CPAD), jnp.float32),
        ],
    )(feats, labs, pt)
    return out.reshape(())


# speedup vs baseline: 2.7222x; 1.0498x over previous
import functools

import jax
import jax.numpy as jnp
from jax import lax
from jax.experimental import pallas as pl
from jax.experimental.pallas import tpu as pltpu

_BLK = 16384
_CPAD = 32


def _loss_body(nblk, c, f_ref, l_ref, pt_ref, out_ref, acc):
    bi = pl.program_id(0)
    ji = pl.program_id(1)
    step = bi * nblk + ji
    nsteps = pl.num_programs(0) * pl.num_programs(1)

    @pl.when(step == 0)
    def _init():
        acc[...] = jnp.zeros_like(acc)

    f = f_ref[0]
    labs = l_ref[0]

    klass = lax.broadcasted_iota(jnp.int32, (_CPAD, _BLK), 0)
    m = (klass == labs).astype(jnp.float32)

    rowsq = jnp.sum(f * f, axis=0, keepdims=True)
    g = jnp.concatenate(
        [f, rowsq, jnp.ones_like(rowsq)], axis=0)
    acc[...] += lax.dot_general(
        g, m, (((1,), (1,)), ((), ())),
        preferred_element_type=jnp.float32)

    @pl.when(step == nsteps - 1)
    def _finish():
        s = acc[0:c, :]
        q = acc[c:c + 1, :]
        n = acc[c + 1:c + 2, :]
        pt = pt_ref[...]
        ps = jnp.sum(pt * s, axis=0, keepdims=True)
        pp = jnp.sum(pt * pt, axis=0, keepdims=True)
        present = n > 0.0
        denom = jnp.where(present, n, 1.0) * jnp.float32(c)
        term = jnp.where(present, (q + n * pp - 2.0 * ps) / denom, 0.0)
        loss = jnp.sum(term) / jnp.sum(present.astype(jnp.float32))
        out_ref[0, 0] = loss


def kernel(features, labels, prototypes):
    b, c, h, w = features.shape
    ncls = prototypes.shape[0]
    npix = h * w
    nblk = npix // _BLK

    feats = features.reshape(b, c, npix)
    labs = labels.astype(jnp.int32).reshape(b, 1, npix)
    pt = jnp.zeros((c, _CPAD), jnp.float32).at[:, :ncls].set(prototypes.T)

    out = pl.pallas_call(
        functools.partial(_loss_body, nblk, c),
        grid=(b, nblk),
        in_specs=[
            pl.BlockSpec((1, c, _BLK), lambda i, j: (i, 0, j)),
            pl.BlockSpec((1, 1, _BLK), lambda i, j: (i, 0, j)),
            pl.BlockSpec((c, _CPAD), lambda i, j: (0, 0)),
        ],
        out_specs=pl.BlockSpec(memory_space=pltpu.SMEM),
        out_shape=jax.ShapeDtypeStruct((1, 1), jnp.float32),
        scratch_shapes=[
            pltpu.VMEM((c + 2, _CPAD), jnp.float32),
        ],
    )(feats, labs, pt)
    return out.reshape(())


# speedup vs baseline: 7.9081x; 2.9050x over previous
import functools

import jax
import jax.numpy as jnp
from jax import lax
from jax.experimental import pallas as pl
from jax.experimental.pallas import tpu as pltpu

_CPAD = 32


def _loss_body(c, h, w, f_ref, l_ref, pt_ref, out_ref, acc):
    step = pl.program_id(0)
    nsteps = pl.num_programs(0)

    @pl.when(step == 0)
    def _init():
        acc[...] = jnp.zeros_like(acc)

    f = f_ref[0].reshape(c, h * w)
    labs = l_ref[0].reshape(1, h * w)

    klass = lax.broadcasted_iota(jnp.int32, (_CPAD, h * w), 0)
    m = (klass == labs).astype(jnp.float32)

    rowsq = jnp.sum(f * f, axis=0, keepdims=True)
    g = jnp.concatenate(
        [f, rowsq, jnp.ones_like(rowsq)], axis=0)
    acc[...] += lax.dot_general(
        g, m, (((1,), (1,)), ((), ())),
        preferred_element_type=jnp.float32)

    @pl.when(step == nsteps - 1)
    def _finish():
        s = acc[0:c, :]
        q = acc[c:c + 1, :]
        n = acc[c + 1:c + 2, :]
        pt = pt_ref[...]
        ps = jnp.sum(pt * s, axis=0, keepdims=True)
        pp = jnp.sum(pt * pt, axis=0, keepdims=True)
        present = n > 0.0
        denom = jnp.where(present, n, 1.0) * jnp.float32(c)
        term = jnp.where(present, (q + n * pp - 2.0 * ps) / denom, 0.0)
        loss = jnp.sum(term) / jnp.sum(present.astype(jnp.float32))
        out_ref[0, 0] = loss


def kernel(features, labels, prototypes):
    b, c, h, w = features.shape
    ncls = prototypes.shape[0]

    labs = labels.astype(jnp.int32).reshape(b, 1, h, w)
    pt = jnp.zeros((c, _CPAD), jnp.float32).at[:, :ncls].set(prototypes.T)

    out = pl.pallas_call(
        functools.partial(_loss_body, c, h, w),
        grid=(b,),
        in_specs=[
            pl.BlockSpec((1, c, h, w), lambda i: (i, 0, 0, 0)),
            pl.BlockSpec((1, 1, h, w), lambda i: (i, 0, 0, 0)),
            pl.BlockSpec((c, _CPAD), lambda i: (0, 0)),
        ],
        out_specs=pl.BlockSpec(memory_space=pltpu.SMEM),
        out_shape=jax.ShapeDtypeStruct((1, 1), jnp.float32),
        scratch_shapes=[
            pltpu.VMEM((c + 2, _CPAD), jnp.float32),
        ],
    )(features, labs, pt)
    return out.reshape(())
